# baseline (speedup 1.0000x reference)
import functools

import jax
import jax.numpy as jnp
from jax import lax
from jax.experimental import pallas as pl
from jax.experimental.pallas import tpu as pltpu
from jax.experimental.pallas import tpu_sc as plsc

_F = 64
_NSPECIES = 14
_DIMS = (1, 3, 5)
_C = 32


def _pair_table(leq, k):
    t = leq.reshape(_NSPECIES, _F * k)
    w = _F * k
    ta = jnp.broadcast_to(t[:, None, :], (_NSPECIES, _NSPECIES, w))
    tb = jnp.broadcast_to(t[None, :, :], (_NSPECIES, _NSPECIES, w))
    return jnp.concatenate([ta, tb], axis=-1).reshape(_NSPECIES * _NSPECIES, 2 * w)


def _gather3(zp, t0, t1, t2):
    info = plsc.get_sparse_core_info()
    nc, ns = info.num_cores, info.num_subcores
    nw = nc * ns
    nch = zp.shape[1]
    bw = nch * _C
    npair = nw * bw
    d0, d1, d2 = (2 * _F * k for k in _DIMS)

    @functools.partial(
        pl.kernel,
        mesh=plsc.VectorSubcoreMesh(core_axis_name="c", subcore_axis_name="s"),
        compiler_params=pltpu.CompilerParams(use_tc_tiling_on_sc=True),
        out_type=[
            jax.ShapeDtypeStruct((npair, d0), jnp.float32),
            jax.ShapeDtypeStruct((npair, d1), jnp.float32),
            jax.ShapeDtypeStruct((npair, d2), jnp.float32),
        ],
        scratch_types=[
            pltpu.VMEM((nch, _C), jnp.int32),
            pltpu.VMEM((2, _C, d0), jnp.float32),
            pltpu.VMEM((2, _C, d1), jnp.float32),
            pltpu.VMEM((2, _C, d2), jnp.float32),
            pltpu.SemaphoreType.DMA,
            pltpu.SemaphoreType.DMA,
        ],
    )
    def k(zp_hbm, t0_hbm, t1_hbm, t2_hbm, o0_hbm, o1_hbm, o2_hbm,
          idx_v, r0, r1, r2, sem0, sem1):
        wid = lax.axis_index("s") * nc + lax.axis_index("c")
        base = wid * bw
        pltpu.sync_copy(zp_hbm.at[wid], idx_v)
        sems = (sem0, sem1)

        def start(ci, b):
            idx = idx_v.at[ci]
            return (
                pltpu.async_copy(t0_hbm.at[idx], r0.at[b], sems[b]),
                pltpu.async_copy(t1_hbm.at[idx], r1.at[b], sems[b]),
                pltpu.async_copy(t2_hbm.at[idx], r2.at[b], sems[b]),
            )

        def drain_store(ci, b):
            pltpu.make_async_copy(t0_hbm.at[idx_v.at[ci]], r0.at[b], sems[b]).wait()
            pltpu.make_async_copy(t1_hbm.at[idx_v.at[ci]], r1.at[b], sems[b]).wait()
            pltpu.make_async_copy(t2_hbm.at[idx_v.at[ci]], r2.at[b], sems[b]).wait()
            off = base + ci * _C
            pltpu.sync_copy(r0.at[b], o0_hbm.at[pl.ds(off, _C)])
            pltpu.sync_copy(r1.at[b], o1_hbm.at[pl.ds(off, _C)])
            pltpu.sync_copy(r2.at[b], o2_hbm.at[pl.ds(off, _C)])

        start(0, 0)

        def body(g, carry):
            for b in (0, 1):
                ci = 2 * g + b
                nxt = ci + 1

                @pl.when(nxt < nch)
                def _():
                    start(nxt, (b + 1) % 2)

                drain_store(ci, b)
            return carry

        lax.fori_loop(0, nch // 2, body, 0)

    return k(zp, t0, t1, t2)


def kernel(Z, leq0, leq1, leq2):
    N = Z.shape[0]
    zpair = Z.astype(jnp.int32).reshape(-1, 2)
    zidx = zpair[:, 0] * _NSPECIES + zpair[:, 1]
    info = plsc.get_sparse_core_info()
    nw = info.num_cores * info.num_subcores
    zp = zidx.reshape(nw, -1, _C)
    t0 = _pair_table(leq0, _DIMS[0])
    t1 = _pair_table(leq1, _DIMS[1])
    t2 = _pair_table(leq2, _DIMS[2])
    o0, o1, o2 = _gather3(zp, t0, t1, t2)
    return (
        o0.reshape(N, _F, 1),
        o1.reshape(N, _F, 3),
        o2.reshape(N, _F, 5),
        jnp.zeros((N, _F, 7), jnp.float32),
        jnp.zeros((N, _F, 9), jnp.float32),
        jnp.zeros((N, _F, 11), jnp.float32),
    )
